# baseline (speedup 1.0000x reference)
import functools

import jax
import jax.numpy as jnp
import numpy as np
from jax.experimental import pallas as pl
from jax.experimental.pallas import tpu as pltpu

DIM = 96
HEADS = 8
HD = DIM // HEADS


def _pool_mats(hw):
    mats = []
    idx = np.arange(hw)
    for k in (3, 5, 7):
        a = (np.abs(idx[:, None] - idx[None, :]) <= k // 2).astype(np.float32)
        mats.append(jnp.asarray(a))
    return mats


def _bdot(a, b, dims):
    return jax.lax.dot_general(a.astype(jnp.bfloat16), b.astype(jnp.bfloat16),
                               dims, preferred_element_type=jnp.float32)


def _preproc_kernel(xt_ref, y_ref, p3_ref, p5_ref, p7_ref, wq_ref, wk_ref,
                    wv_ref, lnw_ref, lnb_ref, qt_ref, kt_ref, vt_ref):
    c = DIM
    y3 = y_ref[0]
    acc = None
    hi = jax.lax.Precision.HIGHEST
    for p_ref, k in ((p3_ref, 3), (p5_ref, 5), (p7_ref, 7)):
        p = p_ref[...]
        zw = jax.lax.dot_general(y3, p, (((2,), (0,)), ((), ())),
                                 preferred_element_type=jnp.float32,
                                 precision=hi)
        pb = jnp.broadcast_to(p[None], (c,) + p.shape)
        z = jax.lax.dot_general(pb, zw, (((2,), (1,)), ((0,), (0,))),
                                preferred_element_type=jnp.float32,
                                precision=hi)
        z = z / float(k * k)
        acc = z if acc is None else acc + z
    z = acc.reshape(c, -1)
    mu = jnp.mean(z, axis=0, keepdims=True)
    var = jnp.mean((z - mu) * (z - mu), axis=0, keepdims=True)
    yn = (z - mu) / jnp.sqrt(var + 1e-5)
    yn = yn * lnw_ref[...] + lnb_ref[...]
    x3 = xt_ref[0]
    cdims = (((1,), (0,)), ((), ()))
    qt_ref[0] = _bdot(wq_ref[...], x3, cdims).astype(jnp.bfloat16)
    kt_ref[0] = _bdot(wk_ref[...], yn, cdims).astype(jnp.bfloat16)
    vt_ref[0] = _bdot(wv_ref[...], yn, cdims).astype(jnp.bfloat16)


def _radix_select_thresholds(key, k1, k2, ones):
    rows = key.shape[0]
    t0 = jnp.zeros((rows, 1), jnp.uint32)
    kf1 = jnp.float32(k1)
    kf2 = jnp.float32(k2)
    one = jnp.float32(1)
    zero = jnp.float32(0)
    cdims = (((1,), (0,)), ((), ()))

    def body(i, carry):
        t1, t2 = carry
        bit = (31 - i).astype(jnp.uint32)
        b = jnp.uint32(1) << bit
        c1 = t1 | b
        c2 = t2 | b
        m1 = jnp.where(key >= c1, one, zero)
        m2 = jnp.where(key >= c2, one, zero)
        cnt1 = _bdot(m1, ones, cdims)
        cnt2 = _bdot(m2, ones, cdims)
        return (jnp.where(cnt1 >= kf1, c1, t1), jnp.where(cnt2 >= kf2, c2, t2))

    return jax.lax.fori_loop(0, 32, body, (t0, t0), unroll=True)


def _attn_kernel(qt_ref, kt_ref, vt_ref, wp_ref, bp_ref, a1_ref, a2_ref,
                 o_ref, *, k1, k2):
    h = pl.program_id(1)
    qh = qt_ref[0, 0]
    kh = kt_ref[0, 0]
    vh = vt_ref[0, 0]
    scale = HD ** (-0.5)
    attn = jax.lax.dot_general(qh, kh, (((0,), (0,)), ((), ())),
                               preferred_element_type=jnp.float32) * scale
    m = jnp.max(attn, axis=1, keepdims=True)
    bits = jax.lax.bitcast_convert_type(attn, jnp.uint32)
    sign = jnp.uint32(0x80000000)
    key = jnp.where(bits >= sign, ~bits, bits | sign)
    ones = jnp.ones((attn.shape[1], 1), jnp.float32)
    t1, t2 = _radix_select_thresholds(key, k1, k2, ones)
    e = jnp.exp(attn - m)
    zero = jnp.zeros_like(e)
    sm1 = jnp.where(key >= t1, e, zero)
    sm2 = jnp.where(key >= t2, e, zero)
    rdims = (((1,), (0,)), ((), ()))
    hi = jax.lax.Precision.HIGHEST
    s1 = jax.lax.dot_general(sm1, ones, rdims,
                             preferred_element_type=jnp.float32, precision=hi)
    s2 = jax.lax.dot_general(sm2, ones, rdims,
                             preferred_element_type=jnp.float32, precision=hi)
    sm1 = sm1 / s1
    sm2 = sm2 / s2
    kdims = (((1,), (1,)), ((), ()))
    out1 = _bdot(vh, sm1, kdims)
    out2 = _bdot(vh, sm2, kdims)
    outh = out1 * a1_ref[0, 0] + out2 * a2_ref[0, 0]
    proj = _bdot(wp_ref[0], outh, (((1,), (0,)), ((), ())))

    @pl.when(h == 0)
    def _init():
        o_ref[0] = proj + bp_ref[...]

    @pl.when(h != 0)
    def _acc():
        o_ref[0] = o_ref[0] + proj


def kernel(x, y, Wq, Wkv, Wp, bp, ln_w, ln_b, a1, a2):
    b, c, hh, ww = x.shape
    n = hh * ww
    heads, hd = HEADS, HD
    p3, p5, p7 = _pool_mats(hh)
    xt = x.reshape(b, c, n)
    wk = Wkv[:c]
    wv = Wkv[c:]
    lnw = ln_w.reshape(c, 1)
    lnb = ln_b.reshape(c, 1)

    qkv_shape = jax.ShapeDtypeStruct((b, c, n), jnp.bfloat16)
    full2 = lambda s: pl.BlockSpec(s, lambda i: (0,) * len(s))
    qt, kt, vt = pl.pallas_call(
        _preproc_kernel,
        grid=(b,),
        in_specs=[
            pl.BlockSpec((1, c, n), lambda i: (i, 0, 0)),
            pl.BlockSpec((1, c, hh, ww), lambda i: (i, 0, 0, 0)),
            full2((hh, ww)), full2((hh, ww)), full2((hh, ww)),
            full2((c, c)), full2((c, c)), full2((c, c)),
            full2((c, 1)), full2((c, 1)),
        ],
        out_specs=[pl.BlockSpec((1, c, n), lambda i: (i, 0, 0))] * 3,
        out_shape=[qkv_shape] * 3,
    )(xt, y, p3, p5, p7, Wq, wk, wv, lnw, lnb)

    qt4 = qt.reshape(b, heads, hd, n)
    kt4 = kt.reshape(b, heads, hd, n)
    vt4 = vt.reshape(b, heads, hd, n)
    wp4 = Wp.reshape(c, heads, hd).transpose(1, 0, 2)

    k1 = int(n / 2)
    k2 = int(n / 3)
    full2g = lambda s: pl.BlockSpec(s, lambda i, j: (0,) * len(s))
    o = pl.pallas_call(
        functools.partial(_attn_kernel, k1=k1, k2=k2),
        grid=(b, heads),
        in_specs=[
            pl.BlockSpec((1, 1, hd, n), lambda i, j: (i, j, 0, 0)),
            pl.BlockSpec((1, 1, hd, n), lambda i, j: (i, j, 0, 0)),
            pl.BlockSpec((1, 1, hd, n), lambda i, j: (i, j, 0, 0)),
            pl.BlockSpec((1, c, hd), lambda i, j: (j, 0, 0)),
            full2g((c, 1)),
            full2g((1, 1)), full2g((1, 1)),
        ],
        out_specs=pl.BlockSpec((1, c, n), lambda i, j: (i, 0, 0)),
        out_shape=jax.ShapeDtypeStruct((b, c, n), jnp.float32),
        compiler_params=pltpu.CompilerParams(
            dimension_semantics=("arbitrary", "arbitrary")),
    )(qt4, kt4, vt4, wp4, bp.reshape(c, 1), a1.reshape(1, 1),
      a2.reshape(1, 1))

    return o.reshape(b, c, hh, ww)


# speedup vs baseline: 1.2230x; 1.2230x over previous
import functools

import jax
import jax.numpy as jnp
import numpy as np
from jax.experimental import pallas as pl
from jax.experimental.pallas import tpu as pltpu

DIM = 96
HEADS = 8
HD = DIM // HEADS


def _pool_mats(hw):
    mats = []
    idx = np.arange(hw)
    for k in (3, 5, 7):
        a = (np.abs(idx[:, None] - idx[None, :]) <= k // 2).astype(np.float32)
        mats.append(jnp.asarray(a))
    return mats


def _bdot(a, b, dims):
    return jax.lax.dot_general(a.astype(jnp.bfloat16), b.astype(jnp.bfloat16),
                               dims, preferred_element_type=jnp.float32)


def _preproc_kernel(xt_ref, y_ref, p3_ref, p5_ref, p7_ref, wq_ref, wk_ref,
                    wv_ref, lnw_ref, lnb_ref, qt_ref, kt_ref, vt_ref):
    c = DIM
    y3 = y_ref[0]
    acc = None
    hi = jax.lax.Precision.HIGHEST
    for p_ref, k in ((p3_ref, 3), (p5_ref, 5), (p7_ref, 7)):
        p = p_ref[...]
        zw = jax.lax.dot_general(y3, p, (((2,), (0,)), ((), ())),
                                 preferred_element_type=jnp.float32,
                                 precision=hi)
        pb = jnp.broadcast_to(p[None], (c,) + p.shape)
        z = jax.lax.dot_general(pb, zw, (((2,), (1,)), ((0,), (0,))),
                                preferred_element_type=jnp.float32,
                                precision=hi)
        z = z / float(k * k)
        acc = z if acc is None else acc + z
    z = acc.reshape(c, -1)
    mu = jnp.mean(z, axis=0, keepdims=True)
    var = jnp.mean((z - mu) * (z - mu), axis=0, keepdims=True)
    yn = (z - mu) / jnp.sqrt(var + 1e-5)
    yn = yn * lnw_ref[...] + lnb_ref[...]
    x3 = xt_ref[0]
    cdims = (((1,), (0,)), ((), ()))
    qt_ref[0] = _bdot(wq_ref[...], x3, cdims).astype(jnp.bfloat16)
    kt_ref[0] = _bdot(wk_ref[...], yn, cdims).astype(jnp.bfloat16)
    vt_ref[0] = _bdot(wv_ref[...], yn, cdims).astype(jnp.bfloat16)


def _radix_select_thresholds(key, k1, k2):
    rshape = key.shape[:-1] + (1,)
    t0 = jnp.zeros(rshape, jnp.uint32)

    def body(i, carry):
        t1, t2 = carry
        bit = (31 - i).astype(jnp.uint32)
        b = jnp.uint32(1) << bit
        c1 = t1 | b
        c2 = t2 | b
        cnt1 = jnp.sum((key >= c1).astype(jnp.int32), axis=-1, keepdims=True)
        cnt2 = jnp.sum((key >= c2).astype(jnp.int32), axis=-1, keepdims=True)
        return (jnp.where(cnt1 >= k1, c1, t1), jnp.where(cnt2 >= k2, c2, t2))

    return jax.lax.fori_loop(0, 32, body, (t0, t0), unroll=True)


def _attn_kernel(qt_ref, kt_ref, vt_ref, wp_ref, bp_ref, a1_ref, a2_ref,
                 o_ref, *, k1, k2):
    g = pl.program_id(1)
    qh = qt_ref[0]
    kh = kt_ref[0]
    vh = vt_ref[0]
    scale = HD ** (-0.5)
    bdims = (((1,), (1,)), ((0,), (0,)))
    attn = jax.lax.dot_general(qh, kh, bdims,
                               preferred_element_type=jnp.float32) * scale
    m = jnp.max(attn, axis=-1, keepdims=True)
    bits = jax.lax.bitcast_convert_type(attn, jnp.uint32)
    sign = jnp.uint32(0x80000000)
    key = jnp.where(bits >= sign, ~bits, bits | sign)
    t1, t2 = _radix_select_thresholds(key, k1, k2)
    e = jnp.exp(attn - m)
    zero = jnp.zeros_like(e)
    sm1 = jnp.where(key >= t1, e, zero)
    sm2 = jnp.where(key >= t2, e, zero)
    s1 = jnp.sum(sm1, axis=-1, keepdims=True)
    s2 = jnp.sum(sm2, axis=-1, keepdims=True)
    sm1 = sm1 / s1
    sm2 = sm2 / s2
    kdims = (((2,), (2,)), ((0,), (0,)))
    out1 = _bdot(vh, sm1, kdims)
    out2 = _bdot(vh, sm2, kdims)
    outh = out1 * a1_ref[0, 0] + out2 * a2_ref[0, 0]
    proj3 = _bdot(wp_ref[...], outh, (((2,), (1,)), ((0,), (0,))))
    proj = jnp.sum(proj3, axis=0)

    @pl.when(g == 0)
    def _init():
        o_ref[0] = proj + bp_ref[...]

    @pl.when(g != 0)
    def _acc():
        o_ref[0] = o_ref[0] + proj


def kernel(x, y, Wq, Wkv, Wp, bp, ln_w, ln_b, a1, a2):
    b, c, hh, ww = x.shape
    n = hh * ww
    heads, hd = HEADS, HD
    p3, p5, p7 = _pool_mats(hh)
    xt = x.reshape(b, c, n)
    wk = Wkv[:c]
    wv = Wkv[c:]
    lnw = ln_w.reshape(c, 1)
    lnb = ln_b.reshape(c, 1)

    qkv_shape = jax.ShapeDtypeStruct((b, c, n), jnp.bfloat16)
    full2 = lambda s: pl.BlockSpec(s, lambda i: (0,) * len(s))
    qt, kt, vt = pl.pallas_call(
        _preproc_kernel,
        grid=(b,),
        in_specs=[
            pl.BlockSpec((1, c, n), lambda i: (i, 0, 0)),
            pl.BlockSpec((1, c, hh, ww), lambda i: (i, 0, 0, 0)),
            full2((hh, ww)), full2((hh, ww)), full2((hh, ww)),
            full2((c, c)), full2((c, c)), full2((c, c)),
            full2((c, 1)), full2((c, 1)),
        ],
        out_specs=[pl.BlockSpec((1, c, n), lambda i: (i, 0, 0))] * 3,
        out_shape=[qkv_shape] * 3,
    )(xt, y, p3, p5, p7, Wq, wk, wv, lnw, lnb)

    qt4 = qt.reshape(b, heads, hd, n)
    kt4 = kt.reshape(b, heads, hd, n)
    vt4 = vt.reshape(b, heads, hd, n)
    wp4 = Wp.reshape(c, heads, hd).transpose(1, 0, 2)

    k1 = int(n / 2)
    k2 = int(n / 3)
    hp = 2
    full2g = lambda s: pl.BlockSpec(s, lambda i, j: (0,) * len(s))
    o = pl.pallas_call(
        functools.partial(_attn_kernel, k1=k1, k2=k2),
        grid=(b, heads // hp),
        in_specs=[
            pl.BlockSpec((1, hp, hd, n), lambda i, j: (i, j, 0, 0)),
            pl.BlockSpec((1, hp, hd, n), lambda i, j: (i, j, 0, 0)),
            pl.BlockSpec((1, hp, hd, n), lambda i, j: (i, j, 0, 0)),
            pl.BlockSpec((hp, c, hd), lambda i, j: (j, 0, 0)),
            full2g((c, 1)),
            full2g((1, 1)), full2g((1, 1)),
        ],
        out_specs=pl.BlockSpec((1, c, n), lambda i, j: (i, 0, 0)),
        out_shape=jax.ShapeDtypeStruct((b, c, n), jnp.float32),
        compiler_params=pltpu.CompilerParams(
            dimension_semantics=("arbitrary", "arbitrary")),
    )(qt4, kt4, vt4, wp4, bp.reshape(c, 1), a1.reshape(1, 1),
      a2.reshape(1, 1))

    return o.reshape(b, c, hh, ww)


# speedup vs baseline: 1.5083x; 1.2332x over previous
import functools

import jax
import jax.numpy as jnp
import numpy as np
from jax.experimental import pallas as pl
from jax.experimental.pallas import tpu as pltpu

DIM = 96
HEADS = 8
HD = DIM // HEADS


def _pool_mats(hw):
    mats = []
    idx = np.arange(hw)
    for k in (3, 5, 7):
        a = (np.abs(idx[:, None] - idx[None, :]) <= k // 2).astype(np.float32)
        mats.append(jnp.asarray(a))
    return mats


def _bdot(a, b, dims):
    return jax.lax.dot_general(a.astype(jnp.bfloat16), b.astype(jnp.bfloat16),
                               dims, preferred_element_type=jnp.float32)


def _preproc_kernel(xt_ref, y_ref, p3_ref, p5_ref, p7_ref, wq_ref, wk_ref,
                    wv_ref, lnw_ref, lnb_ref, qt_ref, kt_ref, vt_ref):
    c = DIM
    y3 = y_ref[0]
    acc = None
    hi = jax.lax.Precision.HIGHEST
    for p_ref, k in ((p3_ref, 3), (p5_ref, 5), (p7_ref, 7)):
        p = p_ref[...]
        zw = jax.lax.dot_general(y3, p, (((2,), (0,)), ((), ())),
                                 preferred_element_type=jnp.float32,
                                 precision=hi)
        pb = jnp.broadcast_to(p[None], (c,) + p.shape)
        z = jax.lax.dot_general(pb, zw, (((2,), (1,)), ((0,), (0,))),
                                preferred_element_type=jnp.float32,
                                precision=hi)
        z = z / float(k * k)
        acc = z if acc is None else acc + z
    z = acc.reshape(c, -1)
    mu = jnp.mean(z, axis=0, keepdims=True)
    var = jnp.mean((z - mu) * (z - mu), axis=0, keepdims=True)
    yn = (z - mu) / jnp.sqrt(var + 1e-5)
    yn = yn * lnw_ref[...] + lnb_ref[...]
    x3 = xt_ref[0]
    cdims = (((1,), (0,)), ((), ()))
    qt_ref[0] = _bdot(wq_ref[...], x3, cdims).astype(jnp.bfloat16)
    kt_ref[0] = _bdot(wk_ref[...], yn, cdims).astype(jnp.bfloat16)
    vt_ref[0] = _bdot(wv_ref[...], yn, cdims).astype(jnp.bfloat16)


def _radix_select_thresholds(key, k1, k2):
    rshape = key.shape[:-2] + (1, key.shape[-1])
    t0 = jnp.zeros(rshape, jnp.uint32)

    def body(i, carry):
        t1, t2 = carry
        bit = (31 - i).astype(jnp.uint32)
        b = jnp.uint32(1) << bit
        c1 = t1 | b
        c2 = t2 | b
        cnt1 = jnp.sum((key >= c1).astype(jnp.int32), axis=-2, keepdims=True)
        cnt2 = jnp.sum((key >= c2).astype(jnp.int32), axis=-2, keepdims=True)
        return (jnp.where(cnt1 >= k1, c1, t1), jnp.where(cnt2 >= k2, c2, t2))

    return jax.lax.fori_loop(0, 32, body, (t0, t0), unroll=True)


def _attn_kernel(qt_ref, kt_ref, vt_ref, wp_ref, bp_ref, a1_ref, a2_ref,
                 o_ref, *, k1, k2):
    g = pl.program_id(1)
    qh = qt_ref[0]
    kh = kt_ref[0]
    vh = vt_ref[0]
    scale = HD ** (-0.5)
    bdims = (((1,), (1,)), ((0,), (0,)))
    attn = jax.lax.dot_general(kh, qh, bdims,
                               preferred_element_type=jnp.float32) * scale
    m = jnp.max(attn, axis=-2, keepdims=True)
    bits = jax.lax.bitcast_convert_type(attn, jnp.uint32)
    sign = jnp.uint32(0x80000000)
    key = jnp.where(bits >= sign, ~bits, bits | sign)
    t1, t2 = _radix_select_thresholds(key, k1, k2)
    e = jnp.exp(attn - m)
    zero = jnp.zeros_like(e)
    sm1 = jnp.where(key >= t1, e, zero)
    sm2 = jnp.where(key >= t2, e, zero)
    s1 = jnp.sum(sm1, axis=-2, keepdims=True)
    s2 = jnp.sum(sm2, axis=-2, keepdims=True)
    sm1 = sm1 / s1
    sm2 = sm2 / s2
    kdims = (((2,), (1,)), ((0,), (0,)))
    out1 = _bdot(vh, sm1, kdims)
    out2 = _bdot(vh, sm2, kdims)
    outh = out1 * a1_ref[0, 0] + out2 * a2_ref[0, 0]
    proj3 = _bdot(wp_ref[...], outh, (((2,), (1,)), ((0,), (0,))))
    proj = jnp.sum(proj3, axis=0)

    @pl.when(g == 0)
    def _init():
        o_ref[0] = proj + bp_ref[...]

    @pl.when(g != 0)
    def _acc():
        o_ref[0] = o_ref[0] + proj


def kernel(x, y, Wq, Wkv, Wp, bp, ln_w, ln_b, a1, a2):
    b, c, hh, ww = x.shape
    n = hh * ww
    heads, hd = HEADS, HD
    p3, p5, p7 = _pool_mats(hh)
    xt = x.reshape(b, c, n)
    wk = Wkv[:c]
    wv = Wkv[c:]
    lnw = ln_w.reshape(c, 1)
    lnb = ln_b.reshape(c, 1)

    qkv_shape = jax.ShapeDtypeStruct((b, c, n), jnp.bfloat16)
    full2 = lambda s: pl.BlockSpec(s, lambda i: (0,) * len(s))
    qt, kt, vt = pl.pallas_call(
        _preproc_kernel,
        grid=(b,),
        in_specs=[
            pl.BlockSpec((1, c, n), lambda i: (i, 0, 0)),
            pl.BlockSpec((1, c, hh, ww), lambda i: (i, 0, 0, 0)),
            full2((hh, ww)), full2((hh, ww)), full2((hh, ww)),
            full2((c, c)), full2((c, c)), full2((c, c)),
            full2((c, 1)), full2((c, 1)),
        ],
        out_specs=[pl.BlockSpec((1, c, n), lambda i: (i, 0, 0))] * 3,
        out_shape=[qkv_shape] * 3,
    )(xt, y, p3, p5, p7, Wq, wk, wv, lnw, lnb)

    qt4 = qt.reshape(b, heads, hd, n)
    kt4 = kt.reshape(b, heads, hd, n)
    vt4 = vt.reshape(b, heads, hd, n)
    wp4 = Wp.reshape(c, heads, hd).transpose(1, 0, 2)

    k1 = int(n / 2)
    k2 = int(n / 3)
    hp = 2
    full2g = lambda s: pl.BlockSpec(s, lambda i, j: (0,) * len(s))
    o = pl.pallas_call(
        functools.partial(_attn_kernel, k1=k1, k2=k2),
        grid=(b, heads // hp),
        in_specs=[
            pl.BlockSpec((1, hp, hd, n), lambda i, j: (i, j, 0, 0)),
            pl.BlockSpec((1, hp, hd, n), lambda i, j: (i, j, 0, 0)),
            pl.BlockSpec((1, hp, hd, n), lambda i, j: (i, j, 0, 0)),
            pl.BlockSpec((hp, c, hd), lambda i, j: (j, 0, 0)),
            full2g((c, 1)),
            full2g((1, 1)), full2g((1, 1)),
        ],
        out_specs=pl.BlockSpec((1, c, n), lambda i, j: (i, 0, 0)),
        out_shape=jax.ShapeDtypeStruct((b, c, n), jnp.float32),
        compiler_params=pltpu.CompilerParams(
            dimension_semantics=("arbitrary", "arbitrary")),
    )(qt4, kt4, vt4, wp4, bp.reshape(c, 1), a1.reshape(1, 1),
      a2.reshape(1, 1))

    return o.reshape(b, c, hh, ww)


# speedup vs baseline: 1.6269x; 1.0787x over previous
import functools

import jax
import jax.numpy as jnp
import numpy as np
from jax.experimental import pallas as pl
from jax.experimental.pallas import tpu as pltpu

DIM = 96
HEADS = 8
HD = DIM // HEADS


def _pool_mats(hw):
    mats = []
    idx = np.arange(hw)
    for k in (3, 5, 7):
        a = (np.abs(idx[:, None] - idx[None, :]) <= k // 2).astype(np.float32)
        mats.append(jnp.asarray(a))
    return mats


def _bdot(a, b, dims):
    return jax.lax.dot_general(a.astype(jnp.bfloat16), b.astype(jnp.bfloat16),
                               dims, preferred_element_type=jnp.float32)


def _preproc_kernel(xt_ref, y_ref, p3_ref, p5_ref, p7_ref, wq_ref, wk_ref,
                    wv_ref, lnw_ref, lnb_ref, qt_ref, kt_ref, vt_ref):
    c = DIM
    y3 = y_ref[0]
    acc = None
    hi = jax.lax.Precision.HIGHEST
    for p_ref, k in ((p3_ref, 3), (p5_ref, 5), (p7_ref, 7)):
        p = p_ref[...]
        zw = jax.lax.dot_general(y3, p, (((2,), (0,)), ((), ())),
                                 preferred_element_type=jnp.float32,
                                 precision=hi)
        pb = jnp.broadcast_to(p[None], (c,) + p.shape)
        z = jax.lax.dot_general(pb, zw, (((2,), (1,)), ((0,), (0,))),
                                preferred_element_type=jnp.float32,
                                precision=hi)
        z = z / float(k * k)
        acc = z if acc is None else acc + z
    z = acc.reshape(c, -1)
    mu = jnp.mean(z, axis=0, keepdims=True)
    var = jnp.mean((z - mu) * (z - mu), axis=0, keepdims=True)
    yn = (z - mu) / jnp.sqrt(var + 1e-5)
    yn = yn * lnw_ref[...] + lnb_ref[...]
    x3 = xt_ref[0]
    cdims = (((1,), (0,)), ((), ()))
    qt_ref[0] = _bdot(wq_ref[...], x3, cdims).astype(jnp.bfloat16)
    kt_ref[0] = _bdot(wk_ref[...], yn, cdims).astype(jnp.bfloat16)
    vt_ref[0] = _bdot(wv_ref[...], yn, cdims).astype(jnp.bfloat16)


def _radix_select_thresholds(key, k1, k2):
    rshape = key.shape[:-2] + (1, key.shape[-1])
    t0 = jnp.zeros(rshape, jnp.uint32)

    def body(i, carry):
        t1, t2 = carry
        bit = (31 - i).astype(jnp.uint32)
        b = jnp.uint32(1) << bit
        c1 = t1 | b
        c2 = t2 | b
        cnt1 = jnp.sum((key >= c1).astype(jnp.int32), axis=-2, keepdims=True)
        cnt2 = jnp.sum((key >= c2).astype(jnp.int32), axis=-2, keepdims=True)
        return (jnp.where(cnt1 >= k1, c1, t1), jnp.where(cnt2 >= k2, c2, t2))

    return jax.lax.fori_loop(0, 32, body, (t0, t0), unroll=True)


def _attn_kernel(qt_ref, kt_ref, vt_ref, wp_ref, bp_ref, a1_ref, a2_ref,
                 o_ref, *, k1, k2):
    g = pl.program_id(1)
    qh = qt_ref[0]
    kh = kt_ref[0]
    vh = vt_ref[0]
    scale = HD ** (-0.5)
    bdims = (((1,), (1,)), ((0,), (0,)))
    attn = jax.lax.dot_general(kh, qh, bdims,
                               preferred_element_type=jnp.float32) * scale
    m = jnp.max(attn, axis=-2, keepdims=True)
    bits = jax.lax.bitcast_convert_type(attn, jnp.uint32)
    sign = jnp.uint32(0x80000000)
    key = jnp.where(bits >= sign, ~bits, bits | sign)
    t1, t2 = _radix_select_thresholds(key, k1, k2)
    e = jnp.exp(attn - m)
    zero = jnp.zeros_like(e)
    sm1 = jnp.where(key >= t1, e, zero)
    sm2 = jnp.where(key >= t2, e, zero)
    s1 = jnp.sum(sm1, axis=-2, keepdims=True)
    s2 = jnp.sum(sm2, axis=-2, keepdims=True)
    sm1 = sm1 / s1
    sm2 = sm2 / s2
    kdims = (((2,), (1,)), ((0,), (0,)))
    out1 = _bdot(vh, sm1, kdims)
    out2 = _bdot(vh, sm2, kdims)
    outh = out1 * a1_ref[0, 0] + out2 * a2_ref[0, 0]
    proj3 = _bdot(wp_ref[...], outh, (((2,), (1,)), ((0,), (0,))))
    proj = jnp.sum(proj3, axis=0)

    @pl.when(g == 0)
    def _init():
        o_ref[0] = proj + bp_ref[...]

    @pl.when(g != 0)
    def _acc():
        o_ref[0] = o_ref[0] + proj


def kernel(x, y, Wq, Wkv, Wp, bp, ln_w, ln_b, a1, a2):
    b, c, hh, ww = x.shape
    n = hh * ww
    heads, hd = HEADS, HD
    p3, p5, p7 = _pool_mats(hh)
    xt = x.reshape(b, c, n)
    wk = Wkv[:c]
    wv = Wkv[c:]
    lnw = ln_w.reshape(c, 1)
    lnb = ln_b.reshape(c, 1)

    qkv_shape = jax.ShapeDtypeStruct((b, c, n), jnp.bfloat16)
    full2 = lambda s: pl.BlockSpec(s, lambda i: (0,) * len(s))
    qt, kt, vt = pl.pallas_call(
        _preproc_kernel,
        grid=(b,),
        in_specs=[
            pl.BlockSpec((1, c, n), lambda i: (i, 0, 0)),
            pl.BlockSpec((1, c, hh, ww), lambda i: (i, 0, 0, 0)),
            full2((hh, ww)), full2((hh, ww)), full2((hh, ww)),
            full2((c, c)), full2((c, c)), full2((c, c)),
            full2((c, 1)), full2((c, 1)),
        ],
        out_specs=[pl.BlockSpec((1, c, n), lambda i: (i, 0, 0))] * 3,
        out_shape=[qkv_shape] * 3,
    )(xt, y, p3, p5, p7, Wq, wk, wv, lnw, lnb)

    qt4 = qt.reshape(b, heads, hd, n)
    kt4 = kt.reshape(b, heads, hd, n)
    vt4 = vt.reshape(b, heads, hd, n)
    wp4 = Wp.reshape(c, heads, hd).transpose(1, 0, 2)

    k1 = int(n / 2)
    k2 = int(n / 3)
    hp = 4
    full2g = lambda s: pl.BlockSpec(s, lambda i, j: (0,) * len(s))
    o = pl.pallas_call(
        functools.partial(_attn_kernel, k1=k1, k2=k2),
        grid=(b, heads // hp),
        in_specs=[
            pl.BlockSpec((1, hp, hd, n), lambda i, j: (i, j, 0, 0)),
            pl.BlockSpec((1, hp, hd, n), lambda i, j: (i, j, 0, 0)),
            pl.BlockSpec((1, hp, hd, n), lambda i, j: (i, j, 0, 0)),
            pl.BlockSpec((hp, c, hd), lambda i, j: (j, 0, 0)),
            full2g((c, 1)),
            full2g((1, 1)), full2g((1, 1)),
        ],
        out_specs=pl.BlockSpec((1, c, n), lambda i, j: (i, 0, 0)),
        out_shape=jax.ShapeDtypeStruct((b, c, n), jnp.float32),
        compiler_params=pltpu.CompilerParams(
            dimension_semantics=("arbitrary", "arbitrary")),
    )(qt4, kt4, vt4, wp4, bp.reshape(c, 1), a1.reshape(1, 1),
      a2.reshape(1, 1))

    return o.reshape(b, c, hh, ww)
